# Initial kernel scaffold; baseline (speedup 1.0000x reference)
import functools

import jax
import jax.numpy as jnp
from jax import lax
from jax.experimental import pallas as pl
from jax.experimental.pallas import tpu as pltpu
from jax.experimental.pallas import tpu_sc as plsc

EMBED_D = 32
NC, NS = 2, 16
NW = NC * NS
CHUNK = 128
CPG = 10
GROUP = CHUNK * CPG
NG = 20
BPW = GROUP * NG
B_TOTAL = NW * BPW


def _gather_body(idx_hbm, table_hbm, out_hbm, idx_v, rows0, rows1, sem0, sem1):
    wid = lax.axis_index("s") * NC + lax.axis_index("c")
    base = wid * BPW
    rows = (rows0, rows1)
    sems = (sem0, sem1)

    pltpu.sync_copy(idx_hbm.at[wid], idx_v)

    def issue(g, b):
        for c in range(CPG):
            pltpu.async_copy(
                table_hbm.at[idx_v.at[g * CPG + c]],
                rows[b].at[pl.ds(c * CHUNK, CHUNK)],
                sems[b],
            )

    def drain(b):
        pltpu.make_async_copy(out_hbm.at[pl.ds(0, GROUP)], rows[b], sems[b]).wait()

    issue(0, 0)
    issue(1, 1)

    def body(i, carry):
        for b in (0, 1):
            g = 2 * i + b
            drain(b)
            pltpu.sync_copy(rows[b], out_hbm.at[pl.ds(base + g * GROUP, GROUP)])
            nxt = g + 2

            @pl.when(nxt < NG)
            def _():
                issue(nxt, b)

        return carry

    lax.fori_loop(0, NG // 2, body, 0)


@functools.partial(
    pl.kernel,
    mesh=plsc.VectorSubcoreMesh(core_axis_name="c", subcore_axis_name="s"),
    out_type=jax.ShapeDtypeStruct((B_TOTAL, EMBED_D), jnp.float32),
    scratch_types=[
        pltpu.VMEM((NG * CPG, CHUNK), jnp.int32),
        pltpu.VMEM((GROUP, EMBED_D), jnp.float32),
        pltpu.VMEM((GROUP, EMBED_D), jnp.float32),
        pltpu.SemaphoreType.DMA,
        pltpu.SemaphoreType.DMA,
    ],
)
def _sc_gather(idx_hbm, table_hbm, out_hbm, idx_v, rows0, rows1, sem0, sem1):
    _gather_body(idx_hbm, table_hbm, out_hbm, idx_v, rows0, rows1, sem0, sem1)


@jax.jit
def kernel(x, table):
    batch, hist = x.shape
    xf = x.reshape(NW, NG * CPG, CHUNK).astype(jnp.int32)
    out = _sc_gather(xf, table)
    return out.reshape(batch, hist, EMBED_D)


# speedup vs baseline: 1.1137x; 1.1137x over previous
import functools

import jax
import jax.numpy as jnp
from jax import lax
from jax.experimental import pallas as pl
from jax.experimental.pallas import tpu as pltpu
from jax.experimental.pallas import tpu_sc as plsc

EMBED_D = 32
NC, NS = 2, 16
NW = NC * NS
CHUNK = 128
CPG = 10
GROUP = CHUNK * CPG
NG = 20
BPW = GROUP * NG
B_TOTAL = NW * BPW


def _gather_body(idx_hbm, table_hbm, out_hbm, idx_v, rows0, rows1, sem0, sem1):
    wid = lax.axis_index("s") * NC + lax.axis_index("c")
    base = wid * BPW
    rows = (rows0, rows1)
    sems = (sem0, sem1)

    pltpu.sync_copy(idx_hbm.at[wid], idx_v)

    def issue(g, b):
        for c in range(CPG):
            pltpu.async_copy(
                table_hbm.at[idx_v.at[g * CPG + c]],
                rows[b].at[pl.ds(c * CHUNK, CHUNK)],
                sems[b],
            )

    def drain(b):
        pltpu.make_async_copy(out_hbm.at[pl.ds(0, GROUP)], rows[b], sems[b]).wait()

    issue(0, 0)
    issue(1, 1)

    def body(i, carry):
        for b in (0, 1):
            g = 2 * i + b
            drain(b)
            pltpu.sync_copy(rows[b], out_hbm.at[pl.ds(base + g * GROUP, GROUP)])
            nxt = g + 2

            @pl.when(nxt < NG)
            def _():
                issue(nxt, b)

        return carry

    lax.fori_loop(0, NG // 2, body, 0)


@functools.partial(
    pl.kernel,
    mesh=plsc.VectorSubcoreMesh(core_axis_name="c", subcore_axis_name="s"),
    out_type=jax.ShapeDtypeStruct((B_TOTAL, EMBED_D), jnp.float32),
    compiler_params=pltpu.CompilerParams(use_tc_tiling_on_sc=False),
    scratch_types=[
        pltpu.VMEM((NG * CPG, CHUNK), jnp.int32),
        pltpu.VMEM((GROUP, EMBED_D), jnp.float32),
        pltpu.VMEM((GROUP, EMBED_D), jnp.float32),
        pltpu.SemaphoreType.DMA,
        pltpu.SemaphoreType.DMA,
    ],
)
def _sc_gather(idx_hbm, table_hbm, out_hbm, idx_v, rows0, rows1, sem0, sem1):
    _gather_body(idx_hbm, table_hbm, out_hbm, idx_v, rows0, rows1, sem0, sem1)


@jax.jit
def kernel(x, table):
    batch, hist = x.shape
    xf = x.reshape(NW, NG * CPG, CHUNK).astype(jnp.int32)
    out = _sc_gather(xf, table)
    return out.reshape(batch, hist, EMBED_D)


# speedup vs baseline: 1.8072x; 1.6227x over previous
import functools

import jax
import jax.numpy as jnp
from jax import lax
from jax.experimental import pallas as pl
from jax.experimental.pallas import tpu as pltpu
from jax.experimental.pallas import tpu_sc as plsc

BATCH = 16384
HIST = 50
EMBED_D = 32
NC, NS = 2, 16
NW = NC * NS
PW = BATCH // NW
RPG = 16
NG = PW // RPG


def _gather_body(idx_hbm, table_hbm, out_hbm, idx_v, buf0, buf1, sem0, sem1):
    wid = lax.axis_index("s") * NC + lax.axis_index("c")
    base = wid * PW
    bufs = (buf0, buf1)
    sems = (sem0, sem1)

    pltpu.sync_copy(idx_hbm.at[wid], idx_v)

    def issue(g, b):
        for r in range(RPG):
            pltpu.async_copy(
                table_hbm.at[idx_v.at[g * RPG + r]],
                bufs[b].at[r],
                sems[b],
            )

    def drain(b):
        pltpu.make_async_copy(out_hbm.at[pl.ds(0, RPG)], bufs[b], sems[b]).wait()

    issue(0, 0)
    issue(1, 1)

    def body(i, carry):
        for b in (0, 1):
            g = 2 * i + b
            drain(b)
            pltpu.sync_copy(bufs[b], out_hbm.at[pl.ds(base + g * RPG, RPG)])
            nxt = g + 2

            @pl.when(nxt < NG)
            def _():
                issue(nxt, b)

        return carry

    lax.fori_loop(0, NG // 2, body, 0)


@functools.partial(
    pl.kernel,
    mesh=plsc.VectorSubcoreMesh(core_axis_name="c", subcore_axis_name="s"),
    out_type=jax.ShapeDtypeStruct((BATCH, HIST, EMBED_D), jnp.float32),
    compiler_params=pltpu.CompilerParams(use_tc_tiling_on_sc=False),
    scratch_types=[
        pltpu.VMEM((PW, HIST), jnp.int32),
        pltpu.VMEM((RPG, HIST, EMBED_D), jnp.float32),
        pltpu.VMEM((RPG, HIST, EMBED_D), jnp.float32),
        pltpu.SemaphoreType.DMA,
        pltpu.SemaphoreType.DMA,
    ],
)
def _sc_gather(idx_hbm, table_hbm, out_hbm, idx_v, buf0, buf1, sem0, sem1):
    _gather_body(idx_hbm, table_hbm, out_hbm, idx_v, buf0, buf1, sem0, sem1)


@jax.jit
def kernel(x, table):
    xf = x.reshape(NW, PW, HIST).astype(jnp.int32)
    return _sc_gather(xf, table)
